# Initial kernel scaffold; baseline (speedup 1.0000x reference)
import functools

import jax
import jax.numpy as jnp
import numpy as np
from jax import lax
from jax.experimental import pallas as pl
from jax.experimental.pallas import tpu as pltpu
from jax.experimental.pallas import tpu_sc as plsc

N = 10000
E = 160000
F = 256
FH = 128
NC = 2
NS = 16
NW = NC * NS

K = 80
EPT = E // NS
STEPS = EPT // K
RPT = N // NS

KD = 40
EPT_D = E // NW
STEPS_D = EPT_D // KD
DW = 16

BM = 1000
GRID = N // BM

_DICT = np.linspace(-4.0, 4.0, 20).astype(np.float32)
_GAMMA = float(0.5 / np.square(_DICT[1] - _DICT[0]))

_MESH = plsc.VectorSubcoreMesh(core_axis_name="c", subcore_axis_name="s")
_PREC = lax.Precision.HIGHEST


@functools.partial(
    pl.kernel,
    mesh=_MESH,
    out_type=jax.ShapeDtypeStruct((NC * N, DW), jnp.float32),
    scratch_types=[
        pltpu.VMEM((STEPS_D, KD), jnp.int32),
        pltpu.VMEM((KD, DW), jnp.float32),
    ],
)
def _deg_sc(cidx_hbm, zeros_hbm, out_hbm, idx_v, ones_v):
    c = lax.axis_index("c")
    s = lax.axis_index("s")
    wid = s * NC + c
    pltpu.sync_copy(cidx_hbm.at[wid], idx_v)
    for r in range(KD):
        ones_v[r, :] = jnp.ones((16,), jnp.float32)

    def _run(acc):
        pltpu.sync_copy(zeros_hbm.at[pl.ds(s * RPT, RPT)],
                        acc.at[pl.ds(s * RPT, RPT)])
        plsc.subcore_barrier()

        def step(i, _):
            pltpu.sync_copy(ones_v, acc.at[idx_v.at[i]], add=True)
            return 0

        lax.fori_loop(0, STEPS_D, step, 0)
        plsc.subcore_barrier()
        pltpu.sync_copy(acc.at[pl.ds(s * RPT, RPT)],
                        out_hbm.at[pl.ds(c * N + s * RPT, RPT)])

    pl.run_scoped(_run, pltpu.VMEM_SHARED((N, DW), jnp.float32))


@functools.partial(
    pl.kernel,
    mesh=_MESH,
    out_type=jax.ShapeDtypeStruct((NC * N, FH), jnp.float32),
    scratch_types=[
        pltpu.VMEM((2 * STEPS, K), jnp.int32),
        pltpu.VMEM((2, K, FH), jnp.float32),
        pltpu.SemaphoreType.DMA,
    ],
)
def _segsum_sc(tab_l_hbm, tab_r_hbm, idx_hbm, zeros_hbm, out_hbm,
               idx_v, rows_v, gsem):
    c = lax.axis_index("c")
    s = lax.axis_index("s")
    pltpu.sync_copy(idx_hbm.at[s], idx_v)

    def _run(acc):
        pltpu.sync_copy(zeros_hbm.at[pl.ds(s * RPT, RPT)],
                        acc.at[pl.ds(s * RPT, RPT)])
        plsc.subcore_barrier()

        def make_step(tab_hbm):
            def step(i, _):
                b = lax.rem(i, 2)
                pltpu.async_copy(tab_hbm.at[idx_v.at[2 * i]], rows_v.at[b],
                                 gsem).wait()
                pltpu.sync_copy(rows_v.at[b], acc.at[idx_v.at[2 * i + 1]],
                                add=True)
                return 0
            return step

        @pl.when(c == 0)
        def _():
            lax.fori_loop(0, STEPS, make_step(tab_l_hbm), 0)

        @pl.when(c == 1)
        def _():
            lax.fori_loop(0, STEPS, make_step(tab_r_hbm), 0)

        plsc.subcore_barrier()
        pltpu.sync_copy(acc.at[pl.ds(s * RPT, RPT)],
                        out_hbm.at[pl.ds(c * N + s * RPT, RPT)])

    pl.run_scoped(_run, pltpu.VMEM_SHARED((N, FH), jnp.float32))


def _dis_block(da, db):
    deg = da + db
    return jnp.where(deg > 0, lax.rsqrt(jnp.maximum(deg, 1e-12)), 0.0)[:, :1]


def _kaf_block(z, at):
    acc = jnp.zeros_like(z)
    for i in range(20):
        acc = acc + at[i:i + 1, :] * jnp.exp(-_GAMMA * (z - float(_DICT[i])) ** 2)
    return acc


def _tc_a_body(x_ref, wi_ref, wr_ref, da_ref, db_ref, yl_ref, yr_ref, r_ref):
    xb = x_ref[...]
    dis = _dis_block(da_ref[...], db_ref[...])
    y = jnp.dot(xb, wi_ref[...], preferred_element_type=jnp.float32,
                precision=_PREC) * dis
    yl_ref[...] = y[:, :FH]
    yr_ref[...] = y[:, FH:]
    r_ref[...] = jnp.dot(xb, wr_ref[...], preferred_element_type=jnp.float32,
                         precision=_PREC)


def _tc_b_body(al_ref, ar_ref, r1_ref, da_ref, db_ref, b_ref, at_ref,
               wi_ref, wr_ref, yl_ref, yr_ref, r2_ref):
    dis = _dis_block(da_ref[...], db_ref[...])
    z = (jnp.concatenate([al_ref[...], ar_ref[...]], axis=1) * dis
         + r1_ref[...] + b_ref[...])
    h = _kaf_block(z, at_ref[...])
    y = jnp.dot(h, wi_ref[...], preferred_element_type=jnp.float32,
                precision=_PREC) * dis
    yl_ref[...] = y[:, :FH]
    yr_ref[...] = y[:, FH:]
    r2_ref[...] = jnp.dot(h, wr_ref[...], preferred_element_type=jnp.float32,
                          precision=_PREC)


def _tc_c_body(al_ref, ar_ref, r2_ref, da_ref, db_ref, b_ref, at_ref, o_ref):
    dis = _dis_block(da_ref[...], db_ref[...])
    z = (jnp.concatenate([al_ref[...], ar_ref[...]], axis=1) * dis
         + r2_ref[...] + b_ref[...])
    o_ref[...] = _kaf_block(z, at_ref[...])


def _rows(i):
    return (i, 0)


def _fixed(i):
    return (0, 0)


_BS_X = pl.BlockSpec((BM, F), _rows)
_BS_H = pl.BlockSpec((BM, FH), _rows)
_BS_W = pl.BlockSpec((F, F), _fixed)
_BS_D = pl.BlockSpec((BM, DW), _rows)
_BS_B = pl.BlockSpec((1, F), _fixed)
_BS_AT = pl.BlockSpec((20, F), _fixed)

_tc_a = pl.pallas_call(
    _tc_a_body,
    grid=(GRID,),
    in_specs=[_BS_X, _BS_W, _BS_W, _BS_D, _BS_D],
    out_specs=[_BS_H, _BS_H, _BS_X],
    out_shape=[
        jax.ShapeDtypeStruct((N, FH), jnp.float32),
        jax.ShapeDtypeStruct((N, FH), jnp.float32),
        jax.ShapeDtypeStruct((N, F), jnp.float32),
    ],
)

_tc_b = pl.pallas_call(
    _tc_b_body,
    grid=(GRID,),
    in_specs=[_BS_H, _BS_H, _BS_X, _BS_D, _BS_D, _BS_B, _BS_AT, _BS_W, _BS_W],
    out_specs=[_BS_H, _BS_H, _BS_X],
    out_shape=[
        jax.ShapeDtypeStruct((N, FH), jnp.float32),
        jax.ShapeDtypeStruct((N, FH), jnp.float32),
        jax.ShapeDtypeStruct((N, F), jnp.float32),
    ],
)

_tc_c = pl.pallas_call(
    _tc_c_body,
    grid=(GRID,),
    in_specs=[_BS_H, _BS_H, _BS_X, _BS_D, _BS_D, _BS_B, _BS_AT],
    out_specs=_BS_X,
    out_shape=jax.ShapeDtypeStruct((N, F), jnp.float32),
)


def kernel(x, edge_index, init_w1, root_w1, bias1, init_w2, root_w2, bias2,
           alpha):
    row = edge_index[0]
    col = edge_index[1]
    r3 = row.reshape(NS, STEPS, 1, K)
    c3 = col.reshape(NS, STEPS, 1, K)
    seg_idx = jnp.concatenate([r3, c3], axis=2).reshape(NS, 2 * STEPS, K)
    deg_idx = col.reshape(NW, STEPS_D, KD)
    zeros_deg = jnp.zeros((N, DW), jnp.float32)
    zeros_acc = jnp.zeros((N, FH), jnp.float32)
    alpha_t = alpha[0].T
    bias1_2d = bias1[None, :]
    bias2_2d = bias2[None, :]

    deg2 = _deg_sc(deg_idx, zeros_deg)
    deg_a, deg_b = deg2[:N], deg2[N:]

    y1l, y1r, r1 = _tc_a(x, init_w1, root_w1, deg_a, deg_b)
    agg1 = _segsum_sc(y1l, y1r, seg_idx, zeros_acc)
    y2l, y2r, r2 = _tc_b(agg1[:N], agg1[N:], r1, deg_a, deg_b, bias1_2d,
                         alpha_t, init_w2, root_w2)
    agg2 = _segsum_sc(y2l, y2r, seg_idx, zeros_acc)
    return _tc_c(agg2[:N], agg2[N:], r2, deg_a, deg_b, bias2_2d, alpha_t)


# speedup vs baseline: 7.1566x; 7.1566x over previous
import functools

import jax
import jax.numpy as jnp
import numpy as np
from jax import lax
from jax.experimental import pallas as pl
from jax.experimental.pallas import tpu as pltpu
from jax.experimental.pallas import tpu_sc as plsc

N = 10000
E = 160000
F = 256
FH = 128
NC = 2
NS = 16
NW = NC * NS

NP = 10240
K = 80
EPT = E // NS
STEPS = EPT // K
RPT = NP // NS

KD = 40
EPT_D = E // NW
STEPS_D = EPT_D // KD
DW = 128

BM = 1000
GRID = N // BM

_DICT = np.linspace(-4.0, 4.0, 20).astype(np.float32)
_GAMMA = float(0.5 / np.square(_DICT[1] - _DICT[0]))

_MESH = plsc.VectorSubcoreMesh(core_axis_name="c", subcore_axis_name="s")
_PREC = lax.Precision.DEFAULT


@functools.partial(
    pl.kernel,
    mesh=_MESH,
    out_type=jax.ShapeDtypeStruct((NC * NP, DW), jnp.float32),
    scratch_types=[
        pltpu.VMEM((STEPS_D, KD), jnp.int32),
        pltpu.VMEM((KD, DW), jnp.float32),
        pltpu.VMEM_SHARED((NP, DW), jnp.float32),
    ],
)
def _deg_sc(cidx_hbm, ones_hbm, zeros_hbm, out_hbm, idx_v, ones_v, acc):
    c = lax.axis_index("c")
    s = lax.axis_index("s")
    wid = s * NC + c
    pltpu.sync_copy(cidx_hbm.at[wid], idx_v)
    pltpu.sync_copy(ones_hbm, ones_v)

    pltpu.sync_copy(zeros_hbm.at[pl.ds(s * RPT, RPT)],
                    acc.at[pl.ds(s * RPT, RPT)])
    plsc.subcore_barrier()

    def step(i, _):
        pltpu.sync_copy(ones_v, acc.at[idx_v.at[i]], add=True)
        return 0

    lax.fori_loop(0, STEPS_D, step, 0)
    plsc.subcore_barrier()
    pltpu.sync_copy(acc.at[pl.ds(s * RPT, RPT)],
                    out_hbm.at[pl.ds(c * NP + s * RPT, RPT)])


@functools.partial(
    pl.kernel,
    mesh=_MESH,
    out_type=jax.ShapeDtypeStruct((NC * NP, FH), jnp.float32),
    scratch_types=[
        pltpu.VMEM((2 * STEPS, K), jnp.int32),
        pltpu.VMEM((1, K, FH), jnp.float32),
        pltpu.VMEM_SHARED((NP, FH), jnp.float32),
        pltpu.SemaphoreType.DMA,
    ],
)
def _segsum_sc(tab_l_hbm, tab_r_hbm, idx_hbm, zeros_hbm, out_hbm,
               idx_v, rows_v, acc, gsem):
    c = lax.axis_index("c")
    s = lax.axis_index("s")
    pltpu.sync_copy(idx_hbm.at[s], idx_v)

    pltpu.sync_copy(zeros_hbm.at[pl.ds(s * RPT, RPT)],
                    acc.at[pl.ds(s * RPT, RPT)])
    plsc.subcore_barrier()

    def make_step(tab_hbm):
        def step(i, _):
            pltpu.async_copy(tab_hbm.at[idx_v.at[2 * i]], rows_v.at[0],
                             gsem).wait()
            pltpu.sync_copy(rows_v.at[0], acc.at[idx_v.at[2 * i + 1]],
                            add=True)
            return 0
        return step

    @pl.when(c == 0)
    def _():
        lax.fori_loop(0, STEPS, make_step(tab_l_hbm), 0)

    @pl.when(c == 1)
    def _():
        lax.fori_loop(0, STEPS, make_step(tab_r_hbm), 0)

    plsc.subcore_barrier()
    pltpu.sync_copy(acc.at[pl.ds(s * RPT, RPT)],
                    out_hbm.at[pl.ds(c * NP + s * RPT, RPT)])


def _dis_block(da, db):
    deg = da + db
    return jnp.where(deg > 0, lax.rsqrt(jnp.maximum(deg, 1e-12)), 0.0)[:, :1]


def _kaf_block(z, at):
    acc = jnp.zeros_like(z)
    for i in range(20):
        acc = acc + at[i:i + 1, :] * jnp.exp(-_GAMMA * (z - float(_DICT[i])) ** 2)
    return acc


def _tc_a_body(x_ref, wi_ref, wr_ref, da_ref, db_ref, yl_ref, yr_ref, r_ref):
    xb = x_ref[...]
    dis = _dis_block(da_ref[...], db_ref[...])
    y = jnp.dot(xb, wi_ref[...], preferred_element_type=jnp.float32,
                precision=_PREC) * dis
    yl_ref[...] = y[:, :FH]
    yr_ref[...] = y[:, FH:]
    r_ref[...] = jnp.dot(xb, wr_ref[...], preferred_element_type=jnp.float32,
                         precision=_PREC)


def _tc_b_body(al_ref, ar_ref, r1_ref, da_ref, db_ref, b_ref, at_ref,
               wi_ref, wr_ref, yl_ref, yr_ref, r2_ref):
    dis = _dis_block(da_ref[...], db_ref[...])
    z = (jnp.concatenate([al_ref[...], ar_ref[...]], axis=1) * dis
         + r1_ref[...] + b_ref[...])
    h = _kaf_block(z, at_ref[...])
    y = jnp.dot(h, wi_ref[...], preferred_element_type=jnp.float32,
                precision=_PREC) * dis
    yl_ref[...] = y[:, :FH]
    yr_ref[...] = y[:, FH:]
    r2_ref[...] = jnp.dot(h, wr_ref[...], preferred_element_type=jnp.float32,
                          precision=_PREC)


def _tc_c_body(al_ref, ar_ref, r2_ref, da_ref, db_ref, b_ref, at_ref, o_ref):
    dis = _dis_block(da_ref[...], db_ref[...])
    z = (jnp.concatenate([al_ref[...], ar_ref[...]], axis=1) * dis
         + r2_ref[...] + b_ref[...])
    o_ref[...] = _kaf_block(z, at_ref[...])


def _rows(i):
    return (i, 0)


def _fixed(i):
    return (0, 0)


_BS_X = pl.BlockSpec((BM, F), _rows)
_BS_H = pl.BlockSpec((BM, FH), _rows)
_BS_W = pl.BlockSpec((F, F), _fixed)
_BS_D = pl.BlockSpec((BM, DW), _rows)
_BS_B = pl.BlockSpec((1, F), _fixed)
_BS_AT = pl.BlockSpec((20, F), _fixed)

_tc_a = pl.pallas_call(
    _tc_a_body,
    grid=(GRID,),
    in_specs=[_BS_X, _BS_W, _BS_W, _BS_D, _BS_D],
    out_specs=[_BS_H, _BS_H, _BS_X],
    out_shape=[
        jax.ShapeDtypeStruct((N, FH), jnp.float32),
        jax.ShapeDtypeStruct((N, FH), jnp.float32),
        jax.ShapeDtypeStruct((N, F), jnp.float32),
    ],
)

_tc_b = pl.pallas_call(
    _tc_b_body,
    grid=(GRID,),
    in_specs=[_BS_H, _BS_H, _BS_X, _BS_D, _BS_D, _BS_B, _BS_AT, _BS_W, _BS_W],
    out_specs=[_BS_H, _BS_H, _BS_X],
    out_shape=[
        jax.ShapeDtypeStruct((N, FH), jnp.float32),
        jax.ShapeDtypeStruct((N, FH), jnp.float32),
        jax.ShapeDtypeStruct((N, F), jnp.float32),
    ],
)

_tc_c = pl.pallas_call(
    _tc_c_body,
    grid=(GRID,),
    in_specs=[_BS_H, _BS_H, _BS_X, _BS_D, _BS_D, _BS_B, _BS_AT],
    out_specs=_BS_X,
    out_shape=jax.ShapeDtypeStruct((N, F), jnp.float32),
)


def kernel(x, edge_index, init_w1, root_w1, bias1, init_w2, root_w2, bias2,
           alpha):
    row = edge_index[0]
    col = edge_index[1]
    r3 = row.reshape(NS, STEPS, 1, K)
    c3 = col.reshape(NS, STEPS, 1, K)
    seg_idx = jnp.concatenate([r3, c3], axis=2).reshape(NS, 2 * STEPS, K)
    deg_idx = col.reshape(NW, STEPS_D, KD)
    zeros_deg = jnp.zeros((NP, DW), jnp.float32)
    zeros_acc = jnp.zeros((NP, FH), jnp.float32)
    alpha_t = alpha[0].T
    bias1_2d = bias1[None, :]
    bias2_2d = bias2[None, :]

    ones_rows = jnp.ones((KD, DW), jnp.float32)
    deg2 = _deg_sc(deg_idx, ones_rows, zeros_deg)
    deg_a, deg_b = deg2[:N], deg2[NP:NP + N]

    y1l, y1r, r1 = _tc_a(x, init_w1, root_w1, deg_a, deg_b)
    agg1 = _segsum_sc(y1l, y1r, seg_idx, zeros_acc)
    y2l, y2r, r2 = _tc_b(agg1[:N], agg1[NP:NP + N], r1, deg_a, deg_b, bias1_2d,
                         alpha_t, init_w2, root_w2)
    agg2 = _segsum_sc(y2l, y2r, seg_idx, zeros_acc)
    return _tc_c(agg2[:N], agg2[NP:NP + N], r2, deg_a, deg_b, bias2_2d, alpha_t)


# speedup vs baseline: 8.4659x; 1.1829x over previous
import functools

import jax
import jax.numpy as jnp
import numpy as np
from jax import lax
from jax.experimental import pallas as pl
from jax.experimental.pallas import tpu as pltpu
from jax.experimental.pallas import tpu_sc as plsc

N = 10000
E = 160000
F = 256
FH = 128
NC = 2
NS = 16
NW = NC * NS

NP = 10240
K = 80
EPT = E // NS
STEPS = EPT // K
RPT = NP // NS

KD = 125
EPT_D = E // NW
STEPS_D = EPT_D // KD
DW = 128

BM = 1000
GRID = N // BM

_DICT = np.linspace(-4.0, 4.0, 20).astype(np.float32)
_GAMMA = float(0.5 / np.square(_DICT[1] - _DICT[0]))

_MESH = plsc.VectorSubcoreMesh(core_axis_name="c", subcore_axis_name="s")
_PREC = lax.Precision.DEFAULT


@functools.partial(
    pl.kernel,
    mesh=_MESH,
    out_type=jax.ShapeDtypeStruct((NC * NP, DW), jnp.float32),
    scratch_types=[
        pltpu.VMEM((STEPS_D, KD), jnp.int32),
        pltpu.VMEM((KD, DW), jnp.float32),
        pltpu.VMEM_SHARED((NP, DW), jnp.float32),
    ],
)
def _deg_sc(cidx_hbm, ones_hbm, zeros_hbm, out_hbm, idx_v, ones_v, acc):
    c = lax.axis_index("c")
    s = lax.axis_index("s")
    wid = s * NC + c
    pltpu.sync_copy(cidx_hbm.at[wid], idx_v)
    pltpu.sync_copy(ones_hbm, ones_v)

    pltpu.sync_copy(zeros_hbm.at[pl.ds(s * RPT, RPT)],
                    acc.at[pl.ds(s * RPT, RPT)])
    plsc.subcore_barrier()

    def step(i, _):
        pltpu.sync_copy(ones_v, acc.at[idx_v.at[i]], add=True)
        return 0

    lax.fori_loop(0, STEPS_D, step, 0)
    plsc.subcore_barrier()
    pltpu.sync_copy(acc.at[pl.ds(s * RPT, RPT)],
                    out_hbm.at[pl.ds(c * NP + s * RPT, RPT)])


@functools.partial(
    pl.kernel,
    mesh=_MESH,
    out_type=jax.ShapeDtypeStruct((NC * NP, FH), jnp.float32),
    scratch_types=[
        pltpu.VMEM((3, 2, K), jnp.int32),
        pltpu.VMEM((2, K, FH), jnp.float32),
        pltpu.VMEM_SHARED((NP, FH), jnp.float32),
        pltpu.SemaphoreType.DMA,
        pltpu.SemaphoreType.DMA,
    ],
)
def _segsum_sc(tab_l_hbm, tab_r_hbm, idx_hbm, zeros_hbm, out_hbm,
               idxr, rows_v, acc, gsem, isem):
    c = lax.axis_index("c")
    s = lax.axis_index("s")
    pltpu.sync_copy(zeros_hbm.at[pl.ds(s * RPT, RPT)],
                    acc.at[pl.ds(s * RPT, RPT)])
    pltpu.async_copy(idx_hbm.at[s, 0], idxr.at[0], isem)
    pltpu.async_copy(idx_hbm.at[s, 1], idxr.at[1], isem)
    plsc.subcore_barrier()

    def run(tab_hbm):
        pltpu.make_async_copy(idx_hbm.at[s, 0], idxr.at[0], isem).wait()
        pltpu.async_copy(tab_hbm.at[idxr.at[0, 0]], rows_v.at[0], gsem)

        def step(i, _):
            b = lax.rem(i, 2)
            m = lax.rem(i, 3)
            pltpu.make_async_copy(tab_hbm.at[idxr.at[m, 0]], rows_v.at[b],
                                  gsem).wait()

            @pl.when(i + 2 < STEPS)
            def _():
                pltpu.async_copy(idx_hbm.at[s, i + 2],
                                 idxr.at[lax.rem(i + 2, 3)], isem)

            @pl.when(i + 1 < STEPS)
            def _():
                m1 = lax.rem(i + 1, 3)
                pltpu.make_async_copy(idx_hbm.at[s, i + 1], idxr.at[m1],
                                      isem).wait()
                pltpu.async_copy(tab_hbm.at[idxr.at[m1, 0]],
                                 rows_v.at[lax.rem(i + 1, 2)], gsem)

            pltpu.sync_copy(rows_v.at[b], acc.at[idxr.at[m, 1]], add=True)
            return 0

        lax.fori_loop(0, STEPS, step, 0)

    @pl.when(c == 0)
    def _():
        run(tab_l_hbm)

    @pl.when(c == 1)
    def _():
        run(tab_r_hbm)

    plsc.subcore_barrier()
    pltpu.sync_copy(acc.at[pl.ds(s * RPT, RPT)],
                    out_hbm.at[pl.ds(c * NP + s * RPT, RPT)])


def _dis_block(da, db):
    deg = da + db
    return jnp.where(deg > 0, lax.rsqrt(jnp.maximum(deg, 1e-12)), 0.0)[:, :1]


def _kaf_block(z, at):
    acc = jnp.zeros_like(z)
    for i in range(20):
        acc = acc + at[i:i + 1, :] * jnp.exp(-_GAMMA * (z - float(_DICT[i])) ** 2)
    return acc


def _tc_a_body(x_ref, wi_ref, wr_ref, da_ref, db_ref, yl_ref, yr_ref, r_ref):
    xb = x_ref[...]
    dis = _dis_block(da_ref[...], db_ref[...])
    y = jnp.dot(xb, wi_ref[...], preferred_element_type=jnp.float32,
                precision=_PREC) * dis
    yl_ref[...] = y[:, :FH]
    yr_ref[...] = y[:, FH:]
    r_ref[...] = jnp.dot(xb, wr_ref[...], preferred_element_type=jnp.float32,
                         precision=_PREC)


def _tc_b_body(al_ref, ar_ref, r1_ref, da_ref, db_ref, b_ref, at_ref,
               wi_ref, wr_ref, yl_ref, yr_ref, r2_ref):
    dis = _dis_block(da_ref[...], db_ref[...])
    z = (jnp.concatenate([al_ref[...], ar_ref[...]], axis=1) * dis
         + r1_ref[...] + b_ref[...])
    h = _kaf_block(z, at_ref[...])
    y = jnp.dot(h, wi_ref[...], preferred_element_type=jnp.float32,
                precision=_PREC) * dis
    yl_ref[...] = y[:, :FH]
    yr_ref[...] = y[:, FH:]
    r2_ref[...] = jnp.dot(h, wr_ref[...], preferred_element_type=jnp.float32,
                          precision=_PREC)


def _tc_c_body(al_ref, ar_ref, r2_ref, da_ref, db_ref, b_ref, at_ref, o_ref):
    dis = _dis_block(da_ref[...], db_ref[...])
    z = (jnp.concatenate([al_ref[...], ar_ref[...]], axis=1) * dis
         + r2_ref[...] + b_ref[...])
    o_ref[...] = _kaf_block(z, at_ref[...])


def _rows(i):
    return (i, 0)


def _fixed(i):
    return (0, 0)


_BS_X = pl.BlockSpec((BM, F), _rows)
_BS_H = pl.BlockSpec((BM, FH), _rows)
_BS_W = pl.BlockSpec((F, F), _fixed)
_BS_D = pl.BlockSpec((BM, DW), _rows)
_BS_B = pl.BlockSpec((1, F), _fixed)
_BS_AT = pl.BlockSpec((20, F), _fixed)

_tc_a = pl.pallas_call(
    _tc_a_body,
    grid=(GRID,),
    in_specs=[_BS_X, _BS_W, _BS_W, _BS_D, _BS_D],
    out_specs=[_BS_H, _BS_H, _BS_X],
    out_shape=[
        jax.ShapeDtypeStruct((N, FH), jnp.float32),
        jax.ShapeDtypeStruct((N, FH), jnp.float32),
        jax.ShapeDtypeStruct((N, F), jnp.float32),
    ],
)

_tc_b = pl.pallas_call(
    _tc_b_body,
    grid=(GRID,),
    in_specs=[_BS_H, _BS_H, _BS_X, _BS_D, _BS_D, _BS_B, _BS_AT, _BS_W, _BS_W],
    out_specs=[_BS_H, _BS_H, _BS_X],
    out_shape=[
        jax.ShapeDtypeStruct((N, FH), jnp.float32),
        jax.ShapeDtypeStruct((N, FH), jnp.float32),
        jax.ShapeDtypeStruct((N, F), jnp.float32),
    ],
)

_tc_c = pl.pallas_call(
    _tc_c_body,
    grid=(GRID,),
    in_specs=[_BS_H, _BS_H, _BS_X, _BS_D, _BS_D, _BS_B, _BS_AT],
    out_specs=_BS_X,
    out_shape=jax.ShapeDtypeStruct((N, F), jnp.float32),
)


def kernel(x, edge_index, init_w1, root_w1, bias1, init_w2, root_w2, bias2,
           alpha):
    row = edge_index[0]
    col = edge_index[1]
    r4 = row.reshape(NS, STEPS, 1, K)
    c4 = col.reshape(NS, STEPS, 1, K)
    seg_idx = jnp.concatenate([r4, c4], axis=2)
    deg_idx = col.reshape(NW, STEPS_D, KD)
    zeros_deg = jnp.zeros((NP, DW), jnp.float32)
    zeros_acc = jnp.zeros((NP, FH), jnp.float32)
    alpha_t = alpha[0].T
    bias1_2d = bias1[None, :]
    bias2_2d = bias2[None, :]

    ones_rows = jnp.ones((KD, DW), jnp.float32)
    deg2 = _deg_sc(deg_idx, ones_rows, zeros_deg)
    deg_a, deg_b = deg2[:N], deg2[NP:NP + N]

    y1l, y1r, r1 = _tc_a(x, init_w1, root_w1, deg_a, deg_b)
    agg1 = _segsum_sc(y1l, y1r, seg_idx, zeros_acc)
    y2l, y2r, r2 = _tc_b(agg1[:N], agg1[NP:NP + N], r1, deg_a, deg_b, bias1_2d,
                         alpha_t, init_w2, root_w2)
    agg2 = _segsum_sc(y2l, y2r, seg_idx, zeros_acc)
    return _tc_c(agg2[:N], agg2[NP:NP + N], r2, deg_a, deg_b, bias2_2d, alpha_t)


# speedup vs baseline: 8.4799x; 1.0017x over previous
import functools

import jax
import jax.numpy as jnp
import numpy as np
from jax import lax
from jax.experimental import pallas as pl
from jax.experimental.pallas import tpu as pltpu
from jax.experimental.pallas import tpu_sc as plsc

N = 10000
E = 160000
F = 256
FH = 128
NC = 2
NS = 16
NW = NC * NS

NP = 10240
K = 80
EPT = E // NS
STEPS = EPT // K
RPT = NP // NS

KD = 125
EPT_D = E // NW
STEPS_D = EPT_D // KD
DW = 128

BM = 1000
GRID = N // BM

_DICT = np.linspace(-4.0, 4.0, 20).astype(np.float32)
_GAMMA = float(0.5 / np.square(_DICT[1] - _DICT[0]))

_MESH = plsc.VectorSubcoreMesh(core_axis_name="c", subcore_axis_name="s")
_PREC = lax.Precision.DEFAULT


@functools.partial(
    pl.kernel,
    mesh=_MESH,
    out_type=jax.ShapeDtypeStruct((NC * NP, DW), jnp.float32),
    scratch_types=[
        pltpu.VMEM((STEPS_D, KD), jnp.int32),
        pltpu.VMEM((KD, DW), jnp.float32),
        pltpu.VMEM_SHARED((NP, DW), jnp.float32),
    ],
)
def _deg_sc(cidx_hbm, ones_hbm, zeros_hbm, out_hbm, idx_v, ones_v, acc):
    c = lax.axis_index("c")
    s = lax.axis_index("s")
    wid = s * NC + c
    pltpu.sync_copy(cidx_hbm.at[wid], idx_v)
    pltpu.sync_copy(ones_hbm, ones_v)

    pltpu.sync_copy(zeros_hbm.at[pl.ds(s * RPT, RPT)],
                    acc.at[pl.ds(s * RPT, RPT)])
    plsc.subcore_barrier()

    def step(i, _):
        pltpu.sync_copy(ones_v, acc.at[idx_v.at[i]], add=True)
        return 0

    lax.fori_loop(0, STEPS_D, step, 0)
    plsc.subcore_barrier()
    pltpu.sync_copy(acc.at[pl.ds(s * RPT, RPT)],
                    out_hbm.at[pl.ds(c * NP + s * RPT, RPT)])


@functools.partial(
    pl.kernel,
    mesh=_MESH,
    out_type=jax.ShapeDtypeStruct((NC * NP, FH), jnp.float32),
    scratch_types=[
        pltpu.VMEM((4, 2, K), jnp.int32),
        pltpu.VMEM((3, K, FH), jnp.float32),
        pltpu.VMEM_SHARED((NP, FH), jnp.float32),
        pltpu.SemaphoreType.DMA,
        pltpu.SemaphoreType.DMA,
        pltpu.SemaphoreType.DMA,
    ],
)
def _segsum_sc(tab_l_hbm, tab_r_hbm, idx_hbm, zeros_hbm, out_hbm,
               idxr, rows_v, acc, gsem, isem, ssem):
    c = lax.axis_index("c")
    s = lax.axis_index("s")
    pltpu.sync_copy(zeros_hbm.at[pl.ds(s * RPT, RPT)],
                    acc.at[pl.ds(s * RPT, RPT)])
    pltpu.async_copy(idx_hbm.at[s, 0], idxr.at[0], isem)
    pltpu.async_copy(idx_hbm.at[s, 1], idxr.at[1], isem)
    plsc.subcore_barrier()

    def run(tab_hbm):
        pltpu.make_async_copy(idx_hbm.at[s, 0], idxr.at[0], isem).wait()
        pltpu.async_copy(tab_hbm.at[idxr.at[0, 0]], rows_v.at[0], gsem)

        def step(i, _):
            b = lax.rem(i, 3)
            m = lax.rem(i, 4)
            pltpu.make_async_copy(tab_hbm.at[idxr.at[m, 0]], rows_v.at[b],
                                  gsem).wait()

            @pl.when(i >= 2)
            def _():
                pltpu.make_async_copy(rows_v.at[b], acc.at[idxr.at[m, 1]],
                                      ssem).wait()

            @pl.when(i + 2 < STEPS)
            def _():
                pltpu.async_copy(idx_hbm.at[s, i + 2],
                                 idxr.at[lax.rem(i + 2, 4)], isem)

            @pl.when(i + 1 < STEPS)
            def _():
                m1 = lax.rem(i + 1, 4)
                pltpu.make_async_copy(idx_hbm.at[s, i + 1], idxr.at[m1],
                                      isem).wait()
                pltpu.async_copy(tab_hbm.at[idxr.at[m1, 0]],
                                 rows_v.at[lax.rem(i + 1, 3)], gsem)

            pltpu.async_copy(rows_v.at[b], acc.at[idxr.at[m, 1]], ssem,
                             add=True)
            return 0

        lax.fori_loop(0, STEPS, step, 0)
        pltpu.make_async_copy(rows_v.at[0], acc.at[idxr.at[0, 1]], ssem).wait()
        pltpu.make_async_copy(rows_v.at[0], acc.at[idxr.at[0, 1]], ssem).wait()

    @pl.when(c == 0)
    def _():
        run(tab_l_hbm)

    @pl.when(c == 1)
    def _():
        run(tab_r_hbm)

    plsc.subcore_barrier()
    pltpu.sync_copy(acc.at[pl.ds(s * RPT, RPT)],
                    out_hbm.at[pl.ds(c * NP + s * RPT, RPT)])


def _dis_block(da, db):
    deg = da + db
    return jnp.where(deg > 0, lax.rsqrt(jnp.maximum(deg, 1e-12)), 0.0)[:, :1]


def _kaf_block(z, at):
    acc = jnp.zeros_like(z)
    for i in range(20):
        acc = acc + at[i:i + 1, :] * jnp.exp(-_GAMMA * (z - float(_DICT[i])) ** 2)
    return acc


def _tc_a_body(x_ref, wi_ref, wr_ref, da_ref, db_ref, yl_ref, yr_ref, r_ref):
    xb = x_ref[...]
    dis = _dis_block(da_ref[...], db_ref[...])
    y = jnp.dot(xb, wi_ref[...], preferred_element_type=jnp.float32,
                precision=_PREC) * dis
    yl_ref[...] = y[:, :FH]
    yr_ref[...] = y[:, FH:]
    r_ref[...] = jnp.dot(xb, wr_ref[...], preferred_element_type=jnp.float32,
                         precision=_PREC)


def _tc_b_body(al_ref, ar_ref, r1_ref, da_ref, db_ref, b_ref, at_ref,
               wi_ref, wr_ref, yl_ref, yr_ref, r2_ref):
    dis = _dis_block(da_ref[...], db_ref[...])
    z = (jnp.concatenate([al_ref[...], ar_ref[...]], axis=1) * dis
         + r1_ref[...] + b_ref[...])
    h = _kaf_block(z, at_ref[...])
    y = jnp.dot(h, wi_ref[...], preferred_element_type=jnp.float32,
                precision=_PREC) * dis
    yl_ref[...] = y[:, :FH]
    yr_ref[...] = y[:, FH:]
    r2_ref[...] = jnp.dot(h, wr_ref[...], preferred_element_type=jnp.float32,
                          precision=_PREC)


def _tc_c_body(al_ref, ar_ref, r2_ref, da_ref, db_ref, b_ref, at_ref, o_ref):
    dis = _dis_block(da_ref[...], db_ref[...])
    z = (jnp.concatenate([al_ref[...], ar_ref[...]], axis=1) * dis
         + r2_ref[...] + b_ref[...])
    o_ref[...] = _kaf_block(z, at_ref[...])


def _rows(i):
    return (i, 0)


def _fixed(i):
    return (0, 0)


_BS_X = pl.BlockSpec((BM, F), _rows)
_BS_H = pl.BlockSpec((BM, FH), _rows)
_BS_W = pl.BlockSpec((F, F), _fixed)
_BS_D = pl.BlockSpec((BM, DW), _rows)
_BS_B = pl.BlockSpec((1, F), _fixed)
_BS_AT = pl.BlockSpec((20, F), _fixed)

_tc_a = pl.pallas_call(
    _tc_a_body,
    grid=(GRID,),
    in_specs=[_BS_X, _BS_W, _BS_W, _BS_D, _BS_D],
    out_specs=[_BS_H, _BS_H, _BS_X],
    out_shape=[
        jax.ShapeDtypeStruct((N, FH), jnp.float32),
        jax.ShapeDtypeStruct((N, FH), jnp.float32),
        jax.ShapeDtypeStruct((N, F), jnp.float32),
    ],
)

_tc_b = pl.pallas_call(
    _tc_b_body,
    grid=(GRID,),
    in_specs=[_BS_H, _BS_H, _BS_X, _BS_D, _BS_D, _BS_B, _BS_AT, _BS_W, _BS_W],
    out_specs=[_BS_H, _BS_H, _BS_X],
    out_shape=[
        jax.ShapeDtypeStruct((N, FH), jnp.float32),
        jax.ShapeDtypeStruct((N, FH), jnp.float32),
        jax.ShapeDtypeStruct((N, F), jnp.float32),
    ],
)

_tc_c = pl.pallas_call(
    _tc_c_body,
    grid=(GRID,),
    in_specs=[_BS_H, _BS_H, _BS_X, _BS_D, _BS_D, _BS_B, _BS_AT],
    out_specs=_BS_X,
    out_shape=jax.ShapeDtypeStruct((N, F), jnp.float32),
)


def kernel(x, edge_index, init_w1, root_w1, bias1, init_w2, root_w2, bias2,
           alpha):
    row = edge_index[0]
    col = edge_index[1]
    r4 = row.reshape(NS, STEPS, 1, K)
    c4 = col.reshape(NS, STEPS, 1, K)
    seg_idx = jnp.concatenate([r4, c4], axis=2)
    deg_idx = col.reshape(NW, STEPS_D, KD)
    zeros_deg = jnp.zeros((NP, DW), jnp.float32)
    zeros_acc = jnp.zeros((NP, FH), jnp.float32)
    alpha_t = alpha[0].T
    bias1_2d = bias1[None, :]
    bias2_2d = bias2[None, :]

    ones_rows = jnp.ones((KD, DW), jnp.float32)
    deg2 = _deg_sc(deg_idx, ones_rows, zeros_deg)
    deg_a, deg_b = deg2[:N], deg2[NP:NP + N]

    y1l, y1r, r1 = _tc_a(x, init_w1, root_w1, deg_a, deg_b)
    agg1 = _segsum_sc(y1l, y1r, seg_idx, zeros_acc)
    y2l, y2r, r2 = _tc_b(agg1[:N], agg1[NP:NP + N], r1, deg_a, deg_b, bias1_2d,
                         alpha_t, init_w2, root_w2)
    agg2 = _segsum_sc(y2l, y2r, seg_idx, zeros_acc)
    return _tc_c(agg2[:N], agg2[NP:NP + N], r2, deg_a, deg_b, bias2_2d, alpha_t)
